# Initial kernel scaffold; baseline (speedup 1.0000x reference)
import jax
import jax.numpy as jnp
from jax.experimental import pallas as pl


def kernel(x, W_msg, W_act, W_finish, W_effect, W_phase, W_position, W_number, W_place, W_attrib):
    raise NotImplementedError("write your pallas kernel here")


# speedup vs baseline: 2.0111x; 2.0111x over previous
import functools

import jax
import jax.numpy as jnp
from jax import lax
from jax.experimental import pallas as pl
from jax.experimental.pallas import tpu as pltpu
from jax.experimental.pallas import tpu_sc as plsc

_B, _T = 1024, 200
_N = _B * _T
_DIMS = (16, 16, 8, 32, 8, 16, 8, 16, 8)
_NF = len(_DIMS)

_NC, _NS = 2, 16
_NW = _NC * _NS
_NTOK = _N // _NW
_CH = 128
_G = 5
_STEP = _CH * _G
_NSTEP = _NTOK // _STEP


def _sc_body(xt_ref, *rest):
    w_refs = rest[:_NF]
    out_refs = rest[_NF:2 * _NF]
    idx_ref = rest[2 * _NF]
    row_refs = rest[2 * _NF + 1:3 * _NF + 1]
    gsem, wsem = rest[3 * _NF + 1], rest[3 * _NF + 2]

    wid = lax.axis_index("s") * _NC + lax.axis_index("c")
    base = wid * _NTOK

    @pl.loop(0, _NSTEP)
    def _step(s):
        t0 = base + s * _STEP
        pltpu.sync_copy(xt_ref.at[pl.ds(t0 // _CH, _G), :, :], idx_ref)
        handles = []
        for i in range(_NF):
            for g in range(_G):
                handles.append(pltpu.async_copy(
                    w_refs[i].at[idx_ref.at[g, i]],
                    row_refs[i].at[pl.ds(g * _CH, _CH), :],
                    gsem))
        for h in handles:
            h.wait()
        wh = [pltpu.async_copy(row_refs[i],
                               out_refs[i].at[pl.ds(t0, _STEP), :], wsem)
              for i in range(_NF)]
        for h in wh:
            h.wait()


@jax.jit
def kernel(x, W_msg, W_act, W_finish, W_effect, W_phase, W_position,
           W_number, W_place, W_attrib):
    Ws = (W_msg, W_act, W_finish, W_effect, W_phase, W_position,
          W_number, W_place, W_attrib)
    xt = x.reshape(_N // _CH, _CH, _NF).transpose(0, 2, 1)

    mesh = plsc.VectorSubcoreMesh(core_axis_name="c", subcore_axis_name="s",
                                  num_cores=_NC, num_subcores=_NS)
    out_type = [jax.ShapeDtypeStruct((_N, d), jnp.float32) for d in _DIMS]
    scratch = ([pltpu.VMEM((_G, _NF, _CH), jnp.int32)]
               + [pltpu.VMEM((_STEP, d), jnp.float32) for d in _DIMS]
               + [pltpu.SemaphoreType.DMA, pltpu.SemaphoreType.DMA])
    outs = pl.kernel(
        _sc_body,
        out_type=out_type,
        mesh=mesh,
        scratch_types=scratch,
        compiler_params=pltpu.CompilerParams(use_tc_tiling_on_sc=False),
    )(xt, *Ws)
    return tuple(o.reshape(_B, _T, d) for o, d in zip(outs, _DIMS))
